# baseline (speedup 1.0000x reference)
import functools

import jax
import jax.numpy as jnp
from jax import lax
from jax.experimental import pallas as pl
from jax.experimental.pallas import tpu as pltpu
from jax.experimental.pallas import tpu_sc as plsc

L = 16
NC = 2
NS = 16
NW = NC * NS
GPW = 3


def _min_index_of(value_v, best_v, best_i, big):
    cand = jnp.where(best_v == value_v, best_i, jnp.int32(big))
    return -plsc.cummax(-cand)[L - 1]


def _lane_top2(best_v, best_i, sec_v, sec_i, big):
    m1 = plsc.cummax(best_v)[L - 1]
    i1 = _min_index_of(m1, best_v, best_i, big)
    is_w = best_i == jnp.full((L,), i1, jnp.int32)
    scv = jnp.where(is_w, sec_v, best_v)
    sci = jnp.where(is_w, sec_i, best_i)
    m2 = plsc.cummax(scv)[L - 1]
    i2 = _min_index_of(m2, scv, sci, big)
    return m1, i1, m2, i2


def _top2_update(s, ids, best_v, best_i, sec_v, sec_i):
    upd1 = s > best_v
    upd2 = s > sec_v
    nsec_v = jnp.where(upd1, best_v, jnp.where(upd2, s, sec_v))
    nsec_i = jnp.where(upd1, best_i, jnp.where(upd2, ids, sec_i))
    return (jnp.where(upd1, s, best_v), jnp.where(upd1, ids, best_i),
            nsec_v, nsec_i)


def _nms_body(n, n_sl,
              sim_rows_hbm, scores_hbm, thr_hbm, keep_hbm, ret_hbm,
              *scr):
    ids_v = scr[0:3]
    val_v = scr[3:6]
    ret_v = scr[6:9]
    keep_v = scr[9:12]
    rowa_v = scr[12:15]
    rowb_v = scr[15:18]
    thr_v = scr[18:21]
    sema = scr[21:24]
    semb = scr[24:27]
    wid = lax.axis_index("s") * NC + lax.axis_index("c")
    iota = lax.iota(jnp.int32, L)
    lane0 = iota == 0
    neg1 = jnp.full((L,), -1, jnp.int32)
    big = n_sl * L
    zerov = jnp.zeros((L,), jnp.float32)
    zeroi = jnp.zeros((L,), jnp.int32)

    gs = [wid + k * NW for k in range(GPW)]

    def record(k, i, idx, ok):
        m = lane0 & (jnp.full((L,), ok.astype(jnp.int32), jnp.int32) != 0)
        plsc.store_scatter(keep_v[k], [jnp.full((L,), i, jnp.int32)],
                           jnp.full((L,), idx, jnp.int32), mask=m)
        plsc.store_scatter(ret_v[k], [jnp.full((L,), idx, jnp.int32)],
                           jnp.full((L,), 1000.0 - i.astype(jnp.float32),
                                    jnp.float32), mask=m)

    def start_dma(k, idx, buf, sem):
        pltpu.make_async_copy(sim_rows_hbm.at[gs[k] * n + idx],
                              buf[k], sem[k]).start()

    def wait_dma(k, buf, sem):
        pltpu.make_async_copy(sim_rows_hbm.at[gs[k] * n], buf[k],
                              sem[k]).wait()

    def init_top2(k):
        best_v, best_i = val_v[k][pl.ds(0, L)], iota
        sec_v, sec_i = zerov, zeroi
        for j in range(1, n_sl):
            s = val_v[k][pl.ds(j * L, L)]
            best_v, best_i, sec_v, sec_i = _top2_update(
                s, iota + j * L, best_v, best_i, sec_v, sec_i)
        return _lane_top2(best_v, best_i, sec_v, sec_i, big)

    def compact_sweep(k, cnt, c1, c2, hitv, thr_vec):
        c1v = jnp.full((L,), c1, jnp.int32)
        c2v = jnp.full((L,), c2, jnp.int32)
        cntv = jnp.full((L,), cnt, jnp.int32)
        nsteps = (cnt + (2 * L - 1)) // (2 * L)

        def step(j, carry):
            off, best_v, best_i, sec_v, sec_i = carry
            for h in range(2):
                base = (2 * j + h) * L
                sl = pl.ds(base, L)
                ids = ids_v[k][sl]
                vals = val_v[k][sl]
                rv1 = plsc.load_gather(rowa_v[k], [ids])
                rv2 = plsc.load_gather(rowb_v[k], [ids])
                pos = iota + base
                dead = ((rv1 > thr_vec) | (ids == c1v)
                        | (hitv & ((rv2 > thr_vec) | (ids == c2v))))
                alive = (pos < cntv) & ~dead
                s = jnp.where(alive, vals, 0.0)
                best_v, best_i, sec_v, sec_i = _top2_update(
                    s, ids, best_v, best_i, sec_v, sec_i)
                osl = pl.ds(off, L)
                plsc.store_compressed(ids_v[k].at[osl], ids, mask=alive)
                plsc.store_compressed(val_v[k].at[osl], vals, mask=alive)
                off = off + plsc.all_reduce_population_count(alive)[0]
            return (off, best_v, best_i, sec_v, sec_i)

        return lax.fori_loop(0, nsteps, step,
                             (jnp.int32(0), zerov, zeroi, zerov, zeroi))

    for k in range(GPW):
        g = gs[k]
        pltpu.make_async_copy(scores_hbm.at[g], val_v[k], sema[k]).start()
        pltpu.make_async_copy(scores_hbm.at[g], ret_v[k], semb[k]).start()
        pltpu.make_async_copy(thr_hbm.at[g], thr_v[k], sema[k]).start()
        for j in range(n_sl):
            keep_v[k][pl.ds(j * L, L)] = neg1
            ids_v[k][pl.ds(j * L, L)] = iota + j * L
    state0 = []
    for k in range(GPW):
        g = gs[k]
        pltpu.make_async_copy(scores_hbm.at[g], val_v[k], sema[k]).wait()
        pltpu.make_async_copy(thr_hbm.at[g], thr_v[k], sema[k]).wait()
        pltpu.make_async_copy(scores_hbm.at[g], ret_v[k], semb[k]).wait()
        m1, c1, m2, c2 = init_top2(k)
        act0 = m1 != 0.0
        record(k, jnp.int32(0), c1, act0)
        start_dma(k, c1, rowa_v, sema)
        start_dma(k, c2, rowb_v, semb)
        state0 += [act0, jnp.int32(1), c1, c2, m2, jnp.int32(n)]

    def cond(c):
        return c[0] | c[6] | c[12]

    def body(c):
        out = list(c)
        hits, i1s, sweeps = [], [], []
        for k in range(GPW):
            i, c2, m2val = c[6 * k + 1], c[6 * k + 3], c[6 * k + 4]
            thr_s = thr_v[k][...][0]
            wait_dma(k, rowa_v, sema)
            wait_dma(k, rowb_v, semb)
            r1c2 = plsc.load_gather(rowa_v[k],
                                    [jnp.full((L,), c2, jnp.int32)])[0]
            hit = (r1c2 <= thr_s) & (m2val != 0.0)
            record(k, i, c2, hit & (i < n))
            hits.append(hit)
            i1s.append(i + hit.astype(jnp.int32))
        for k in range(GPW):
            c1, c2, cnt = c[6 * k + 2], c[6 * k + 3], c[6 * k + 5]
            hitv = jnp.full((L,), hits[k], jnp.bool_)
            sweeps.append(compact_sweep(k, cnt, c1, c2, hitv,
                                        thr_v[k][...]))
        tops = [_lane_top2(bv, bi, sv, si, big)
                for (_, bv, bi, sv, si) in sweeps]
        for k in range(GPW):
            m1n, nc1, m2n, nc2 = tops[k]
            i1 = i1s[k]
            act = (m1n != 0.0) & (i1 < n)
            record(k, i1, nc1, act)
            start_dma(k, nc1, rowa_v, sema)
            start_dma(k, nc2, rowb_v, semb)
            out[6 * k:6 * k + 6] = [act, i1 + act.astype(jnp.int32),
                                    nc1, nc2, m2n, sweeps[k][0]]
        return tuple(out)

    lax.while_loop(cond, body, tuple(state0))

    for k in range(GPW):
        wait_dma(k, rowa_v, sema)
        wait_dma(k, rowb_v, semb)
        pltpu.make_async_copy(keep_v[k], keep_hbm.at[gs[k]], sema[k]).start()
        pltpu.make_async_copy(ret_v[k], ret_hbm.at[gs[k]], semb[k]).start()
    for k in range(GPW):
        pltpu.make_async_copy(keep_v[k], keep_hbm.at[gs[k]], sema[k]).wait()
        pltpu.make_async_copy(ret_v[k], ret_hbm.at[gs[k]], semb[k]).wait()


@functools.partial(jax.jit, static_argnums=(3, 4, 5))
def _sc_nms(sim_rows, scores_t, thr, bg, n, n_sl):
    mesh = plsc.VectorSubcoreMesh(core_axis_name="c", subcore_axis_name="s",
                                  num_cores=NC, num_subcores=NS)
    body = functools.partial(_nms_body, n, n_sl)
    return pl.kernel(
        body,
        out_type=[jax.ShapeDtypeStruct((bg, n), jnp.int32),
                  jax.ShapeDtypeStruct((bg, n), jnp.float32)],
        mesh=mesh,
        compiler_params=pltpu.CompilerParams(needs_layout_passes=False),
        scratch_types=(
            [pltpu.VMEM((n,), jnp.int32)] * GPW
            + [pltpu.VMEM((n,), jnp.float32)] * GPW
            + [pltpu.VMEM((n,), jnp.float32)] * GPW
            + [pltpu.VMEM((n,), jnp.int32)] * GPW
            + [pltpu.VMEM((n,), jnp.float32)] * GPW
            + [pltpu.VMEM((n,), jnp.float32)] * GPW
            + [pltpu.VMEM((L,), jnp.float32)] * GPW
            + [pltpu.SemaphoreType.DMA] * GPW
            + [pltpu.SemaphoreType.DMA] * GPW
        ),
    )(sim_rows, scores_t, thr)


def kernel(similarity_matrix, scores, threshold):
    B, G, N, _ = similarity_matrix.shape
    bg = B * G
    assert bg == NW * GPW and N % L == 0
    sim_rows = similarity_matrix.reshape(bg * N, N)
    scores_t = jnp.transpose(scores, (0, 2, 1)).reshape(bg, N)
    thr = jnp.broadcast_to(threshold[:, None, None], (B, G, L)).reshape(bg, L)
    keep_flat, ret_flat = _sc_nms(sim_rows, scores_t, thr, bg, N, N // L)
    keep = keep_flat.reshape(B, G, N)
    ret = jnp.transpose(ret_flat.reshape(B, G, N), (0, 2, 1))
    return keep, ret


# speedup vs baseline: 1.2050x; 1.2050x over previous
import functools

import jax
import jax.numpy as jnp
from jax import lax
from jax.experimental import pallas as pl
from jax.experimental.pallas import tpu as pltpu
from jax.experimental.pallas import tpu_sc as plsc

L = 16
NC = 2
NS = 16
NW = NC * NS
GPW = 3


def _min_index_of(value_v, best_v, best_i, big):
    cand = jnp.where(best_v == value_v, best_i, jnp.int32(big))
    return -plsc.cummax(-cand)[L - 1]


def _lane_top2(best_v, best_i, sec_v, sec_i, big):
    m1 = plsc.cummax(best_v)[L - 1]
    i1 = _min_index_of(m1, best_v, best_i, big)
    is_w = best_i == jnp.full((L,), i1, jnp.int32)
    scv = jnp.where(is_w, sec_v, best_v)
    sci = jnp.where(is_w, sec_i, best_i)
    m2 = plsc.cummax(scv)[L - 1]
    i2 = _min_index_of(m2, scv, sci, big)
    return m1, i1, m2, i2


def _nms_body(n, n_sl,
              sim_rows_hbm, scores_hbm, thr_hbm, keep_hbm, ret_hbm,
              *scr):
    ids_v = scr[0:3]
    val_v = scr[3:6]
    ret_v = scr[6:9]
    keep_v = scr[9:12]
    rowa_v = scr[12:15]
    rowb_v = scr[15:18]
    thr_v = scr[18:21]
    sema = scr[21:24]
    semb = scr[24:27]
    wid = lax.axis_index("s") * NC + lax.axis_index("c")
    iota = lax.iota(jnp.int32, L)
    lane0 = iota == 0
    neg1 = jnp.full((L,), -1, jnp.int32)
    big = n_sl * L
    zerov = jnp.zeros((L,), jnp.float32)

    gs = [wid + k * NW for k in range(GPW)]

    def record(k, i, idx):
        plsc.store_scatter(keep_v[k], [jnp.full((L,), i, jnp.int32)],
                           jnp.full((L,), idx, jnp.int32), mask=lane0)
        plsc.store_scatter(ret_v[k], [jnp.full((L,), idx, jnp.int32)],
                           jnp.full((L,), 1000.0 - i.astype(jnp.float32),
                                    jnp.float32), mask=lane0)

    def start_dma(k, idx, buf, sem):
        pltpu.make_async_copy(sim_rows_hbm.at[gs[k] * n + idx],
                              buf[k], sem[k]).start()

    def wait_dma(k, buf, sem):
        pltpu.make_async_copy(sim_rows_hbm.at[gs[k] * n], buf[k],
                              sem[k]).wait()

    def init_top2(k):
        best_v, sec_v = zerov, zerov
        best_i = jnp.zeros((L,), jnp.int32)
        sec_i = jnp.zeros((L,), jnp.int32)
        first = True
        for j in range(n_sl):
            s = val_v[k][pl.ds(j * L, L)]
            lanes = iota + j * L
            if first:
                best_v, best_i, first = s, lanes, False
            else:
                upd1 = s > best_v
                upd2 = s > sec_v
                nsec_v = jnp.where(upd1, best_v, jnp.where(upd2, s, sec_v))
                nsec_i = jnp.where(upd1, best_i, jnp.where(upd2, lanes, sec_i))
                best_v = jnp.where(upd1, s, best_v)
                best_i = jnp.where(upd1, lanes, best_i)
                sec_v, sec_i = nsec_v, nsec_i
        return _lane_top2(best_v, best_i, sec_v, sec_i, big)

    def compact_sweep(k, cnt, c1, c2, hitv, thr_vec):
        c1v = jnp.full((L,), c1, jnp.int32)
        c2v = jnp.full((L,), c2, jnp.int32)
        cntv = jnp.full((L,), cnt, jnp.int32)
        nsteps = (cnt + (2 * L - 1)) // (2 * L)

        def body(j, carry):
            off, best_v, best_i, sec_v, sec_i = carry
            for h in range(2):
                base = (2 * j + h) * L
                sl = pl.ds(base, L)
                ids = ids_v[k][sl]
                vals = val_v[k][sl]
                rv1 = plsc.load_gather(rowa_v[k], [ids])
                rv2 = plsc.load_gather(rowb_v[k], [ids])
                pos = iota + base
                dead = ((rv1 > thr_vec) | (ids == c1v)
                        | (hitv & ((rv2 > thr_vec) | (ids == c2v))))
                alive = (pos < cntv) & ~dead
                s = jnp.where(alive, vals, 0.0)
                upd1 = s > best_v
                upd2 = s > sec_v
                nsec_v = jnp.where(upd1, best_v, jnp.where(upd2, s, sec_v))
                nsec_i = jnp.where(upd1, best_i, jnp.where(upd2, ids, sec_i))
                best_v = jnp.where(upd1, s, best_v)
                best_i = jnp.where(upd1, ids, best_i)
                osl = pl.ds(off, L)
                plsc.store_compressed(ids_v[k].at[osl], ids, mask=alive)
                plsc.store_compressed(val_v[k].at[osl], vals, mask=alive)
                off = off + plsc.all_reduce_population_count(alive)[0]
            return (off, best_v, best_i, nsec_v, nsec_i)

        off, bv, bi, sv, si = lax.fori_loop(
            0, nsteps, body,
            (jnp.int32(0), zerov, jnp.zeros((L,), jnp.int32),
             zerov, jnp.zeros((L,), jnp.int32)))
        m1, i1, m2, i2 = _lane_top2(bv, bi, sv, si, big)
        return off, m1, i1, m2, i2

    for k in range(GPW):
        g = gs[k]
        pltpu.make_async_copy(scores_hbm.at[g], val_v[k], sema[k]).start()
        pltpu.make_async_copy(scores_hbm.at[g], ret_v[k], semb[k]).start()
        pltpu.make_async_copy(thr_hbm.at[g], thr_v[k], sema[k]).start()
        for j in range(n_sl):
            keep_v[k][pl.ds(j * L, L)] = neg1
            ids_v[k][pl.ds(j * L, L)] = iota + j * L
    state0 = []
    for k in range(GPW):
        g = gs[k]
        pltpu.make_async_copy(scores_hbm.at[g], val_v[k], sema[k]).wait()
        pltpu.make_async_copy(thr_hbm.at[g], thr_v[k], sema[k]).wait()
        pltpu.make_async_copy(scores_hbm.at[g], ret_v[k], semb[k]).wait()
        m1, c1, m2, c2 = init_top2(k)
        act0 = m1 != 0.0

        @pl.when(act0)
        def _():
            record(k, jnp.int32(0), c1)

        start_dma(k, c1, rowa_v, sema)
        start_dma(k, c2, rowb_v, semb)
        state0 += [act0, jnp.int32(1), c1, c2, m2, jnp.int32(n)]

    def cond(c):
        return c[0] | c[6] | c[12]

    def body(c):
        out = list(c)
        for k in range(GPW):
            i, c1, c2, m2val, cnt = (c[6 * k + 1], c[6 * k + 2],
                                     c[6 * k + 3], c[6 * k + 4],
                                     c[6 * k + 5])
            thr_vec = thr_v[k][...]
            thr_s = thr_vec[0]
            wait_dma(k, rowa_v, sema)
            wait_dma(k, rowb_v, semb)
            r1c2 = plsc.load_gather(rowa_v[k],
                                    [jnp.full((L,), c2, jnp.int32)])[0]
            hit = (r1c2 <= thr_s) & (m2val != 0.0)
            hitv = jnp.full((L,), hit, jnp.bool_)

            @pl.when(hit & (i < n))
            def _():
                record(k, i, c2)

            i1 = i + hit.astype(jnp.int32)
            ncnt, m1n, nc1, m2n, nc2 = compact_sweep(
                k, cnt, c1, c2, hitv, thr_vec)
            act = (m1n != 0.0) & (i1 < n)

            @pl.when(act)
            def _():
                record(k, i1, nc1)

            start_dma(k, nc1, rowa_v, sema)
            start_dma(k, nc2, rowb_v, semb)
            out[6 * k:6 * k + 6] = [act, i1 + act.astype(jnp.int32),
                                    nc1, nc2, m2n, ncnt]
        return tuple(out)

    lax.while_loop(cond, body, tuple(state0))

    for k in range(GPW):
        wait_dma(k, rowa_v, sema)
        wait_dma(k, rowb_v, semb)
        pltpu.make_async_copy(keep_v[k], keep_hbm.at[gs[k]], sema[k]).start()
        pltpu.make_async_copy(ret_v[k], ret_hbm.at[gs[k]], semb[k]).start()
    for k in range(GPW):
        pltpu.make_async_copy(keep_v[k], keep_hbm.at[gs[k]], sema[k]).wait()
        pltpu.make_async_copy(ret_v[k], ret_hbm.at[gs[k]], semb[k]).wait()


@functools.partial(jax.jit, static_argnums=(3, 4, 5))
def _sc_nms(sim_rows, scores_t, thr, bg, n, n_sl):
    mesh = plsc.VectorSubcoreMesh(core_axis_name="c", subcore_axis_name="s",
                                  num_cores=NC, num_subcores=NS)
    body = functools.partial(_nms_body, n, n_sl)
    return pl.kernel(
        body,
        out_type=[jax.ShapeDtypeStruct((bg, n), jnp.int32),
                  jax.ShapeDtypeStruct((bg, n), jnp.float32)],
        mesh=mesh,
        compiler_params=pltpu.CompilerParams(needs_layout_passes=False),
        scratch_types=(
            [pltpu.VMEM((n,), jnp.int32)] * GPW
            + [pltpu.VMEM((n,), jnp.float32)] * GPW
            + [pltpu.VMEM((n,), jnp.float32)] * GPW
            + [pltpu.VMEM((n,), jnp.int32)] * GPW
            + [pltpu.VMEM((n,), jnp.float32)] * GPW
            + [pltpu.VMEM((n,), jnp.float32)] * GPW
            + [pltpu.VMEM((L,), jnp.float32)] * GPW
            + [pltpu.SemaphoreType.DMA] * GPW
            + [pltpu.SemaphoreType.DMA] * GPW
        ),
    )(sim_rows, scores_t, thr)


def kernel(similarity_matrix, scores, threshold):
    B, G, N, _ = similarity_matrix.shape
    bg = B * G
    assert bg == NW * GPW and N % L == 0
    sim_rows = similarity_matrix.reshape(bg * N, N)
    scores_t = jnp.transpose(scores, (0, 2, 1)).reshape(bg, N)
    thr = jnp.broadcast_to(threshold[:, None, None], (B, G, L)).reshape(bg, L)
    keep_flat, ret_flat = _sc_nms(sim_rows, scores_t, thr, bg, N, N // L)
    keep = keep_flat.reshape(B, G, N)
    ret = jnp.transpose(ret_flat.reshape(B, G, N), (0, 2, 1))
    return keep, ret


# speedup vs baseline: 1.2210x; 1.0132x over previous
import functools

import jax
import jax.numpy as jnp
from jax import lax
from jax.experimental import pallas as pl
from jax.experimental.pallas import tpu as pltpu
from jax.experimental.pallas import tpu_sc as plsc

L = 16
NC = 2
NS = 16
NW = NC * NS
GPW = 3


def _min_index_of(value_v, best_v, best_i, big):
    cand = jnp.where(best_v == value_v, best_i, jnp.int32(big))
    return -plsc.cummax(-cand)[L - 1]


def _lane_top2(best_v, best_i, sec_v, sec_i, big):
    m1 = plsc.cummax(best_v)[L - 1]
    i1 = _min_index_of(m1, best_v, best_i, big)
    is_w = best_i == jnp.full((L,), i1, jnp.int32)
    scv = jnp.where(is_w, sec_v, best_v)
    sci = jnp.where(is_w, sec_i, best_i)
    m2 = plsc.cummax(scv)[L - 1]
    i2 = _min_index_of(m2, scv, sci, big)
    return m1, i1, m2, i2


def _nms_body(n, n_sl,
              sim_rows_hbm, scores_hbm, thr_hbm, keep_hbm, ret_hbm,
              *scr):
    ids_v = scr[0:3]
    val_v = scr[3:6]
    ret_v = scr[6:9]
    keep_v = scr[9:12]
    rowa_v = scr[12:15]
    rowb_v = scr[15:18]
    thr_v = scr[18:21]
    sema = scr[21:24]
    semb = scr[24:27]
    wid = lax.axis_index("s") * NC + lax.axis_index("c")
    iota = lax.iota(jnp.int32, L)
    lane0 = iota == 0
    neg1 = jnp.full((L,), -1, jnp.int32)
    big = n_sl * L
    zerov = jnp.zeros((L,), jnp.float32)

    gs = [wid + k * NW for k in range(GPW)]

    def record(k, i, idx):
        plsc.store_scatter(keep_v[k], [jnp.full((L,), i, jnp.int32)],
                           jnp.full((L,), idx, jnp.int32), mask=lane0)
        plsc.store_scatter(ret_v[k], [jnp.full((L,), idx, jnp.int32)],
                           jnp.full((L,), 1000.0 - i.astype(jnp.float32),
                                    jnp.float32), mask=lane0)

    def start_dma(k, idx, buf, sem):
        pltpu.make_async_copy(sim_rows_hbm.at[gs[k] * n + idx],
                              buf[k], sem[k]).start()

    def wait_dma(k, buf, sem):
        pltpu.make_async_copy(sim_rows_hbm.at[gs[k] * n], buf[k],
                              sem[k]).wait()

    def init_top2(k):
        best_v, sec_v = zerov, zerov
        best_i = jnp.zeros((L,), jnp.int32)
        sec_i = jnp.zeros((L,), jnp.int32)
        first = True
        for j in range(n_sl):
            s = val_v[k][pl.ds(j * L, L)]
            lanes = iota + j * L
            if first:
                best_v, best_i, first = s, lanes, False
            else:
                upd1 = s > best_v
                upd2 = s > sec_v
                nsec_v = jnp.where(upd1, best_v, jnp.where(upd2, s, sec_v))
                nsec_i = jnp.where(upd1, best_i, jnp.where(upd2, lanes, sec_i))
                best_v = jnp.where(upd1, s, best_v)
                best_i = jnp.where(upd1, lanes, best_i)
                sec_v, sec_i = nsec_v, nsec_i
        return _lane_top2(best_v, best_i, sec_v, sec_i, big)

    def compact_sweep(k, cnt, c1, c2, hitv, thr_vec):
        c1v = jnp.full((L,), c1, jnp.int32)
        c2v = jnp.full((L,), c2, jnp.int32)
        cntv = jnp.full((L,), cnt, jnp.int32)
        nslices = (cnt + (L - 1)) // L

        def body(j, carry):
            off, best_v, best_i, sec_v, sec_i = carry
            sl = pl.ds(j * L, L)
            ids = ids_v[k][sl]
            vals = val_v[k][sl]
            rv1 = plsc.load_gather(rowa_v[k], [ids])
            rv2 = plsc.load_gather(rowb_v[k], [ids])
            pos = iota + j * L
            dead = ((rv1 > thr_vec) | (ids == c1v)
                    | (hitv & ((rv2 > thr_vec) | (ids == c2v))))
            alive = (pos < cntv) & ~dead
            s = jnp.where(alive, vals, 0.0)
            upd1 = s > best_v
            upd2 = s > sec_v
            nsec_v = jnp.where(upd1, best_v, jnp.where(upd2, s, sec_v))
            nsec_i = jnp.where(upd1, best_i, jnp.where(upd2, ids, sec_i))
            best_v = jnp.where(upd1, s, best_v)
            best_i = jnp.where(upd1, ids, best_i)
            osl = pl.ds(off, L)
            plsc.store_compressed(ids_v[k].at[osl], ids, mask=alive)
            plsc.store_compressed(val_v[k].at[osl], vals, mask=alive)
            npc = plsc.all_reduce_population_count(alive)[0]
            return (off + npc, best_v, best_i, nsec_v, nsec_i)

        off, bv, bi, sv, si = lax.fori_loop(
            0, nslices, body,
            (jnp.int32(0), zerov, jnp.zeros((L,), jnp.int32),
             zerov, jnp.zeros((L,), jnp.int32)))
        m1, i1, m2, i2 = _lane_top2(bv, bi, sv, si, big)
        return off, m1, i1, m2, i2

    for k in range(GPW):
        g = gs[k]
        pltpu.make_async_copy(scores_hbm.at[g], val_v[k], sema[k]).start()
        pltpu.make_async_copy(scores_hbm.at[g], ret_v[k], semb[k]).start()
        pltpu.make_async_copy(thr_hbm.at[g], thr_v[k], sema[k]).start()
        for j in range(n_sl):
            keep_v[k][pl.ds(j * L, L)] = neg1
            ids_v[k][pl.ds(j * L, L)] = iota + j * L
    state0 = []
    for k in range(GPW):
        g = gs[k]
        pltpu.make_async_copy(scores_hbm.at[g], val_v[k], sema[k]).wait()
        pltpu.make_async_copy(thr_hbm.at[g], thr_v[k], sema[k]).wait()
        pltpu.make_async_copy(scores_hbm.at[g], ret_v[k], semb[k]).wait()
        m1, c1, m2, c2 = init_top2(k)
        act0 = m1 != 0.0

        @pl.when(act0)
        def _():
            record(k, jnp.int32(0), c1)

        start_dma(k, c1, rowa_v, sema)
        start_dma(k, c2, rowb_v, semb)
        state0 += [act0, jnp.int32(1), c1, c2, m2, jnp.int32(n)]

    def cond(c):
        return c[0] | c[6] | c[12]

    def body(c):
        out = list(c)
        for k in range(GPW):
            i, c1, c2, m2val, cnt = (c[6 * k + 1], c[6 * k + 2],
                                     c[6 * k + 3], c[6 * k + 4],
                                     c[6 * k + 5])
            thr_vec = thr_v[k][...]
            thr_s = thr_vec[0]
            wait_dma(k, rowa_v, sema)
            wait_dma(k, rowb_v, semb)
            r1c2 = plsc.load_gather(rowa_v[k],
                                    [jnp.full((L,), c2, jnp.int32)])[0]
            hit = (r1c2 <= thr_s) & (m2val != 0.0)
            hitv = jnp.full((L,), hit, jnp.bool_)

            @pl.when(hit & (i < n))
            def _():
                record(k, i, c2)

            i1 = i + hit.astype(jnp.int32)
            ncnt, m1n, nc1, m2n, nc2 = compact_sweep(
                k, cnt, c1, c2, hitv, thr_vec)
            act = (m1n != 0.0) & (i1 < n)

            @pl.when(act)
            def _():
                record(k, i1, nc1)

            start_dma(k, nc1, rowa_v, sema)
            start_dma(k, nc2, rowb_v, semb)
            out[6 * k:6 * k + 6] = [act, i1 + act.astype(jnp.int32),
                                    nc1, nc2, m2n, ncnt]
        return tuple(out)

    lax.while_loop(cond, body, tuple(state0))

    for k in range(GPW):
        wait_dma(k, rowa_v, sema)
        wait_dma(k, rowb_v, semb)
        pltpu.make_async_copy(keep_v[k], keep_hbm.at[gs[k]], sema[k]).start()
        pltpu.make_async_copy(ret_v[k], ret_hbm.at[gs[k]], semb[k]).start()
    for k in range(GPW):
        pltpu.make_async_copy(keep_v[k], keep_hbm.at[gs[k]], sema[k]).wait()
        pltpu.make_async_copy(ret_v[k], ret_hbm.at[gs[k]], semb[k]).wait()


@functools.partial(jax.jit, static_argnums=(3, 4, 5))
def _sc_nms(sim_rows, scores_t, thr, bg, n, n_sl):
    mesh = plsc.VectorSubcoreMesh(core_axis_name="c", subcore_axis_name="s",
                                  num_cores=NC, num_subcores=NS)
    body = functools.partial(_nms_body, n, n_sl)
    return pl.kernel(
        body,
        out_type=[jax.ShapeDtypeStruct((bg, n), jnp.int32),
                  jax.ShapeDtypeStruct((bg, n), jnp.float32)],
        mesh=mesh,
        compiler_params=pltpu.CompilerParams(needs_layout_passes=False),
        scratch_types=(
            [pltpu.VMEM((n,), jnp.int32)] * GPW
            + [pltpu.VMEM((n,), jnp.float32)] * GPW
            + [pltpu.VMEM((n,), jnp.float32)] * GPW
            + [pltpu.VMEM((n,), jnp.int32)] * GPW
            + [pltpu.VMEM((n,), jnp.float32)] * GPW
            + [pltpu.VMEM((n,), jnp.float32)] * GPW
            + [pltpu.VMEM((L,), jnp.float32)] * GPW
            + [pltpu.SemaphoreType.DMA] * GPW
            + [pltpu.SemaphoreType.DMA] * GPW
        ),
    )(sim_rows, scores_t, thr)


def kernel(similarity_matrix, scores, threshold):
    B, G, N, _ = similarity_matrix.shape
    bg = B * G
    assert bg == NW * GPW and N % L == 0
    sim_rows = similarity_matrix.reshape(bg * N, N)
    scores_t = jnp.transpose(scores, (0, 2, 1)).reshape(bg, N)
    thr = jnp.broadcast_to(threshold[:, None, None], (B, G, L)).reshape(bg, L)
    keep_flat, ret_flat = _sc_nms(sim_rows, scores_t, thr, bg, N, N // L)
    keep = keep_flat.reshape(B, G, N)
    ret = jnp.transpose(ret_flat.reshape(B, G, N), (0, 2, 1))
    return keep, ret
